# baseline (speedup 1.0000x reference)
import functools

import jax
import jax.numpy as jnp
from jax import lax
from jax.experimental import pallas as pl
from jax.experimental.pallas import tpu as pltpu
from jax.experimental.pallas import tpu_sc as plsc

N = 10000
E = 320000
D_IN = 128
H1 = 64
ROW = 50
NC = 2
NS = 16
NW = NC * NS
RPW = E // (NW * ROW)
NPAD = 10240

_sc_mesh = plsc.VectorSubcoreMesh(core_axis_name="c", subcore_axis_name="s")
_sc_params = pltpu.CompilerParams(use_tc_tiling_on_sc=False)


@functools.partial(
    pl.kernel,
    out_type=jax.ShapeDtypeStruct((NC, NPAD), jnp.float32),
    mesh=_sc_mesh,
    compiler_params=_sc_params,
    scratch_types=[
        pltpu.VMEM((640,), jnp.float32),
        pltpu.VMEM((RPW, ROW), jnp.int32),
        pltpu.VMEM((RPW, ROW), jnp.float32),
        pltpu.VMEM_SHARED((NPAD,), jnp.float32),
        pltpu.SemaphoreType.DMA,
    ],
)
def _deg_kernel(dst_hbm, ew_hbm, out_hbm, zbuf, dstv, ewv, acc, sem):
    cid = lax.axis_index("c")
    sid = lax.axis_index("s")
    wid = sid * NC + cid
    zeros = jnp.zeros((16,), jnp.float32)
    for i in range(40):
        zbuf[pl.ds(i * 16, 16)] = zeros
    pltpu.sync_copy(zbuf, acc.at[pl.ds(sid * 640, 640)])
    plsc.subcore_barrier()
    pltpu.sync_copy(dst_hbm.at[wid], dstv)
    pltpu.sync_copy(ew_hbm.at[wid], ewv)

    def fire(j, carry):
        pltpu.async_copy(ewv.at[j], acc.at[dstv.at[j]], sem, add=True)
        return carry

    lax.fori_loop(0, RPW, fire, 0)

    def drain(j, carry):
        pltpu.make_async_copy(ewv.at[j], acc.at[dstv.at[j]], sem).wait()
        return carry

    lax.fori_loop(0, RPW, drain, 0)
    plsc.subcore_barrier()

    @pl.when(sid == 0)
    def _():
        pltpu.sync_copy(acc, out_hbm.at[cid])


def _g_body(x_ref, wt_ref, deg_ref, g_ref):
    dsum = deg_ref[0] + deg_ref[1] + 1.0
    dinv = jnp.where(dsum > 0, lax.rsqrt(dsum), 0.0)
    h = jnp.dot(x_ref[...], wt_ref[...], preferred_element_type=jnp.float32)
    g_ref[...] = dinv[:N] * h


_g_call = pl.pallas_call(
    _g_body,
    out_shape=jax.ShapeDtypeStruct((N, H1), jnp.float32),
)


def _scale_rows(buf, ewv, j):
    offs = [0, 16, 32]
    for off in offs:
        wv = ewv[j, pl.ds(off, 16)]
        for e16 in range(16):
            e = off + e16
            w = wv[e16]
            for k in range(H1 // 16):
                sl = pl.ds(k * 16, 16)
                buf[e, sl] = buf[e, sl] * w
    wv = ewv[j, pl.ds(ROW - 16, 16)]
    for e in range(48, ROW):
        w = wv[e - (ROW - 16)]
        for k in range(H1 // 16):
            sl = pl.ds(k * 16, 16)
            buf[e, sl] = buf[e, sl] * w


@functools.partial(
    pl.kernel,
    out_type=jax.ShapeDtypeStruct((NC, NPAD, H1), jnp.float32),
    mesh=_sc_mesh,
    compiler_params=_sc_params,
    scratch_types=[
        pltpu.VMEM((16, H1), jnp.float32),
        pltpu.VMEM((RPW, ROW), jnp.int32),
        pltpu.VMEM((RPW, ROW), jnp.int32),
        pltpu.VMEM((RPW, ROW), jnp.float32),
        pltpu.VMEM((ROW, H1), jnp.float32),
        pltpu.VMEM((ROW, H1), jnp.float32),
        pltpu.VMEM((ROW, H1), jnp.float32),
        pltpu.VMEM((ROW, H1), jnp.float32),
        pltpu.VMEM_SHARED((NPAD, H1), jnp.float32),
        pltpu.VMEM_SHARED((N, H1), jnp.float32),
        pltpu.SemaphoreType.DMA,
        pltpu.SemaphoreType.DMA,
        pltpu.SemaphoreType.DMA,
        pltpu.SemaphoreType.DMA,
        pltpu.SemaphoreType.DMA,
        pltpu.SemaphoreType.DMA,
        pltpu.SemaphoreType.DMA,
        pltpu.SemaphoreType.DMA,
    ],
)
def _agg_kernel(src_hbm, dst_hbm, ew_hbm, g_hbm, out_hbm,
                zb, srcv, dstv, ewv, b0, b1, b2, b3, acc, g_sp,
                sg0, sg1, sg2, sg3, ss0, ss1, ss2, ss3):
    cid = lax.axis_index("c")
    sid = lax.axis_index("s")
    wid = sid * NC + cid
    zeros = jnp.zeros((16,), jnp.float32)
    for r in range(16):
        for k in range(H1 // 16):
            zb[r, pl.ds(k * 16, 16)] = zeros
    for k in range(40):
        pltpu.sync_copy(zb, acc.at[pl.ds(sid * 640 + k * 16, 16)])
    pltpu.sync_copy(g_hbm.at[pl.ds(sid * 625, 625)], g_sp.at[pl.ds(sid * 625, 625)])
    plsc.subcore_barrier()
    pltpu.sync_copy(src_hbm.at[wid], srcv)
    pltpu.sync_copy(dst_hbm.at[wid], dstv)
    pltpu.sync_copy(ew_hbm.at[wid], ewv)

    bufs = [b0, b1, b2, b3]
    sgs = [sg0, sg1, sg2, sg3]
    sss = [ss0, ss1, ss2, ss3]
    NIT = RPW // 4

    for k in range(3):
        pltpu.async_copy(g_sp.at[srcv.at[k]], bufs[k], sgs[k])

    def body(i, carry):
        for k in range(4):
            r = 4 * i + k
            pltpu.make_async_copy(g_sp.at[srcv.at[r]], bufs[k], sgs[k]).wait()
            _scale_rows(bufs[k], ewv, r)
            pltpu.async_copy(bufs[k], acc.at[dstv.at[r]], sss[k], add=True)
            q = (k + 3) % 4
            nxt = r + 3
            if k == 0:
                @pl.when(i > 0)
                def _():
                    pltpu.make_async_copy(bufs[q], acc.at[dstv.at[0]],
                                          sss[q]).wait()
                pltpu.async_copy(g_sp.at[srcv.at[nxt]], bufs[q], sgs[q])
            else:
                pltpu.make_async_copy(bufs[q], acc.at[dstv.at[0]],
                                      sss[q]).wait()

                @pl.when(i < NIT - 1)
                def _():
                    pltpu.async_copy(g_sp.at[srcv.at[nxt]], bufs[q], sgs[q])
        return carry

    lax.fori_loop(0, NIT, body, 0)
    pltpu.make_async_copy(bufs[3], acc.at[dstv.at[0]], sss[3]).wait()
    plsc.subcore_barrier()
    pltpu.sync_copy(acc.at[pl.ds(sid * 640, 640)],
                    out_hbm.at[cid, pl.ds(sid * 640, 640)])


def _head_body(part_ref, g_ref, deg_ref, bg_ref, gam_ref, bet_ref,
               w1t_ref, b1_ref, w2t_ref, b2_ref, out_ref):
    dsum = deg_ref[0] + deg_ref[1] + 1.0
    dinv = jnp.where(dsum > 0, lax.rsqrt(dsum), 0.0)[:N]
    p = part_ref[0, :N, :] + part_ref[1, :N, :]
    agg = dinv * (p + g_ref[...])
    t = jnp.maximum(agg + bg_ref[...], 0.0)
    mu = jnp.mean(t, axis=1, keepdims=True)
    d = t - mu
    var = jnp.mean(d * d, axis=1, keepdims=True)
    ln = d * lax.rsqrt(var + 1e-5) * gam_ref[...] + bet_ref[...]
    pooled = jnp.sum(ln, axis=0, keepdims=True) * (1.0 / N)
    z = jnp.maximum(
        jnp.dot(pooled, w1t_ref[...], preferred_element_type=jnp.float32)
        + b1_ref[...], 0.0)
    out_ref[...] = (jnp.dot(z, w2t_ref[...], preferred_element_type=jnp.float32)
                    + b2_ref[...])


def _make_head(a_dim):
    return pl.pallas_call(
        _head_body,
        out_shape=jax.ShapeDtypeStruct((1, a_dim), jnp.float32),
    )


def kernel(x, edge_index, edge_weight, W_gcn, b_gcn, gamma, beta, W1, b1, W2, b2):
    src2 = edge_index[0].reshape(NW, RPW, ROW)
    dst2 = edge_index[1].reshape(NW, RPW, ROW)
    ew2 = edge_weight.reshape(NW, RPW, ROW)

    deg_p = _deg_kernel(dst2, ew2)
    deg3 = deg_p.reshape(NC, NPAD, 1)
    g = _g_call(x, W_gcn.T, deg3)
    part = _agg_kernel(src2, dst2, ew2, g)
    head = _make_head(W2.shape[0])
    return head(part, g, deg3, b_gcn.reshape(1, H1), gamma.reshape(1, H1),
                beta.reshape(1, H1), W1.T, b1.reshape(1, -1), W2.T,
                b2.reshape(1, -1))


# speedup vs baseline: 1.2032x; 1.2032x over previous
import functools

import jax
import jax.numpy as jnp
from jax import lax
from jax.experimental import pallas as pl
from jax.experimental.pallas import tpu as pltpu
from jax.experimental.pallas import tpu_sc as plsc

N = 10000
E = 320000
D_IN = 128
H1 = 64
ROW = 100
NC = 2
NS = 16
NW = NC * NS
RPW = E // (NW * ROW)
NPAD = 10240

_sc_mesh = plsc.VectorSubcoreMesh(core_axis_name="c", subcore_axis_name="s")
_sc_params = pltpu.CompilerParams(use_tc_tiling_on_sc=False,
                                  needs_layout_passes=False)


@functools.partial(
    pl.kernel,
    out_type=jax.ShapeDtypeStruct((NC, NPAD), jnp.float32),
    mesh=_sc_mesh,
    compiler_params=_sc_params,
    scratch_types=[
        pltpu.VMEM((640,), jnp.float32),
        pltpu.VMEM((RPW, ROW), jnp.int32),
        pltpu.VMEM((RPW, ROW), jnp.float32),
        pltpu.VMEM_SHARED((NPAD,), jnp.float32),
        pltpu.SemaphoreType.DMA,
    ],
)
def _deg_kernel(ei_hbm, ew_hbm, out_hbm, zbuf, dstv, ewv, acc, sem):
    cid = lax.axis_index("c")
    sid = lax.axis_index("s")
    wid = sid * NC + cid
    zeros = jnp.zeros((16,), jnp.float32)
    for i in range(40):
        zbuf[pl.ds(i * 16, 16)] = zeros
    pltpu.sync_copy(zbuf, acc.at[pl.ds(sid * 640, 640)])
    plsc.subcore_barrier()
    pltpu.sync_copy(ei_hbm.at[1, wid], dstv)
    pltpu.sync_copy(ew_hbm.at[wid], ewv)

    def fire(j, carry):
        pltpu.async_copy(ewv.at[j], acc.at[dstv.at[j]], sem, add=True)
        return carry

    lax.fori_loop(0, RPW, fire, 0)

    def drain(j, carry):
        pltpu.make_async_copy(ewv.at[j], acc.at[dstv.at[j]], sem).wait()
        return carry

    lax.fori_loop(0, RPW, drain, 0)
    plsc.subcore_barrier()

    @pl.when(sid == 0)
    def _():
        pltpu.sync_copy(acc, out_hbm.at[cid])


def _g_body(x_ref, wt_ref, deg_ref, g_ref):
    dsum = deg_ref[0] + deg_ref[1] + 1.0
    dinv = jnp.where(dsum > 0, lax.rsqrt(dsum), 0.0)
    h = jnp.dot(x_ref[...], wt_ref[...], preferred_element_type=jnp.float32)
    g_ref[...] = (dinv[:N] * h).astype(jnp.bfloat16)


_g_call = pl.pallas_call(
    _g_body,
    out_shape=jax.ShapeDtypeStruct((N, H1), jnp.bfloat16),
)


def _scale_rows(buf, ewv, j):
    def scale_one(e, w):
        for k in range(H1 // 32):
            sl = pl.ds(k * 32, 32)
            lo, hi = plsc.unpack(buf[e, sl], format=plsc.PackFormat.INTERLEAVED)
            buf[e, sl] = plsc.pack(lo * w, hi * w,
                                   format=plsc.PackFormat.INTERLEAVED)

    for off in (0, 16, 32, 48, 64, 80):
        wv = ewv[j, pl.ds(off, 16)]
        for e16 in range(16):
            scale_one(off + e16, wv[e16])
    wv = ewv[j, pl.ds(84, 16)]
    for e in range(96, ROW):
        scale_one(e, wv[e - 84])


@functools.partial(
    pl.kernel,
    out_type=jax.ShapeDtypeStruct((NC, NPAD, H1), jnp.bfloat16),
    mesh=_sc_mesh,
    compiler_params=_sc_params,
    scratch_types=[
        pltpu.VMEM((16, H1), jnp.bfloat16),
        pltpu.VMEM((RPW, ROW), jnp.int32),
        pltpu.VMEM((RPW, ROW), jnp.int32),
        pltpu.VMEM((RPW, ROW), jnp.float32),
        pltpu.VMEM((ROW, H1), jnp.bfloat16),
        pltpu.VMEM((ROW, H1), jnp.bfloat16),
        pltpu.VMEM((ROW, H1), jnp.bfloat16),
        pltpu.VMEM((ROW, H1), jnp.bfloat16),
        pltpu.VMEM_SHARED((NPAD, H1), jnp.bfloat16),
        pltpu.SemaphoreType.DMA,
        pltpu.SemaphoreType.DMA,
        pltpu.SemaphoreType.DMA,
        pltpu.SemaphoreType.DMA,
        pltpu.SemaphoreType.DMA,
        pltpu.SemaphoreType.DMA,
        pltpu.SemaphoreType.DMA,
        pltpu.SemaphoreType.DMA,
    ],
)
def _agg_kernel(ei_hbm, ew_hbm, g_hbm, out_hbm,
                zb, srcv, dstv, ewv, b0, b1, b2, b3, acc,
                sg0, sg1, sg2, sg3, ss0, ss1, ss2, ss3):
    cid = lax.axis_index("c")
    sid = lax.axis_index("s")
    wid = sid * NC + cid
    zeros = jnp.zeros((32,), jnp.bfloat16)
    for r in range(16):
        for k in range(H1 // 32):
            zb[r, pl.ds(k * 32, 32)] = zeros
    for k in range(40):
        pltpu.sync_copy(zb, acc.at[pl.ds(sid * 640 + k * 16, 16)])
    plsc.subcore_barrier()
    pltpu.sync_copy(ei_hbm.at[0, wid], srcv)
    pltpu.sync_copy(ei_hbm.at[1, wid], dstv)
    pltpu.sync_copy(ew_hbm.at[wid], ewv)

    bufs = [b0, b1, b2, b3]
    sgs = [sg0, sg1, sg2, sg3]
    sss = [ss0, ss1, ss2, ss3]
    NIT = RPW // 4

    for k in range(3):
        pltpu.async_copy(g_hbm.at[srcv.at[k]], bufs[k], sgs[k])

    def body(i, carry):
        for k in range(4):
            r = 4 * i + k
            pltpu.make_async_copy(g_hbm.at[srcv.at[r]], bufs[k], sgs[k]).wait()
            _scale_rows(bufs[k], ewv, r)
            pltpu.async_copy(bufs[k], acc.at[dstv.at[r]], sss[k], add=True)
            q = (k + 3) % 4
            nxt = r + 3
            if k == 0:
                @pl.when(i > 0)
                def _():
                    pltpu.make_async_copy(bufs[q], acc.at[dstv.at[0]],
                                          sss[q]).wait()
                pltpu.async_copy(g_hbm.at[srcv.at[nxt]], bufs[q], sgs[q])
            else:
                pltpu.make_async_copy(bufs[q], acc.at[dstv.at[0]],
                                      sss[q]).wait()

                @pl.when(i < NIT - 1)
                def _():
                    pltpu.async_copy(g_hbm.at[srcv.at[nxt]], bufs[q], sgs[q])
        return carry

    lax.fori_loop(0, NIT, body, 0)
    pltpu.make_async_copy(bufs[3], acc.at[dstv.at[0]], sss[3]).wait()
    plsc.subcore_barrier()
    pltpu.sync_copy(acc.at[pl.ds(sid * 640, 640)],
                    out_hbm.at[cid, pl.ds(sid * 640, 640)])


def _head_body(part_ref, g_ref, deg_ref, bg_ref, gam_ref, bet_ref,
               w1t_ref, b1_ref, w2t_ref, b2_ref, out_ref):
    dsum = deg_ref[0] + deg_ref[1] + 1.0
    dinv = jnp.where(dsum > 0, lax.rsqrt(dsum), 0.0)[:N]
    p = (part_ref[0, :N, :].astype(jnp.float32)
         + part_ref[1, :N, :].astype(jnp.float32))
    agg = dinv * (p + g_ref[...].astype(jnp.float32))
    t = jnp.maximum(agg + bg_ref[...], 0.0)
    mu = jnp.mean(t, axis=1, keepdims=True)
    d = t - mu
    var = jnp.mean(d * d, axis=1, keepdims=True)
    ln = d * lax.rsqrt(var + 1e-5) * gam_ref[...] + bet_ref[...]
    pooled = jnp.sum(ln, axis=0, keepdims=True) * (1.0 / N)
    z = jnp.maximum(
        jnp.dot(pooled, w1t_ref[...], preferred_element_type=jnp.float32)
        + b1_ref[...], 0.0)
    out_ref[...] = (jnp.dot(z, w2t_ref[...], preferred_element_type=jnp.float32)
                    + b2_ref[...])


def _make_head(a_dim):
    return pl.pallas_call(
        _head_body,
        out_shape=jax.ShapeDtypeStruct((1, a_dim), jnp.float32),
    )


def kernel(x, edge_index, edge_weight, W_gcn, b_gcn, gamma, beta, W1, b1, W2, b2):
    ei4 = edge_index.reshape(2, NW, RPW, ROW)
    ew3 = edge_weight.reshape(NW, RPW, ROW)

    deg_p = _deg_kernel(ei4, ew3)
    deg3 = deg_p.reshape(NC, NPAD, 1)
    g = _g_call(x, W_gcn.T, deg3)
    part = _agg_kernel(ei4, ew3, g)
    head = _make_head(W2.shape[0])
    return head(part, g, deg3, b_gcn.reshape(1, H1), gamma.reshape(1, H1),
                beta.reshape(1, H1), W1.T, b1.reshape(1, -1), W2.T,
                b2.reshape(1, -1))


# speedup vs baseline: 1.6104x; 1.3384x over previous
import functools

import jax
import jax.numpy as jnp
from jax import lax
from jax.experimental import pallas as pl
from jax.experimental.pallas import tpu as pltpu
from jax.experimental.pallas import tpu_sc as plsc

N = 10000
E = 320000
D_IN = 128
H1 = 64
ROW = 100
NC = 2
NS = 16
NW = NC * NS
RPW = E // (NW * ROW)
NPAD = 10240

_sc_mesh = plsc.VectorSubcoreMesh(core_axis_name="c", subcore_axis_name="s")
_sc_params = pltpu.CompilerParams(use_tc_tiling_on_sc=False,
                                  needs_layout_passes=False)


@functools.partial(
    pl.kernel,
    out_type=jax.ShapeDtypeStruct((NC, NPAD), jnp.float32),
    mesh=_sc_mesh,
    compiler_params=_sc_params,
    scratch_types=[
        pltpu.VMEM((640,), jnp.float32),
        pltpu.VMEM((RPW, ROW), jnp.int32),
        pltpu.VMEM((RPW, ROW), jnp.float32),
        pltpu.VMEM_SHARED((NPAD,), jnp.float32),
        pltpu.SemaphoreType.DMA,
    ],
)
def _deg_kernel(ei_hbm, ew_hbm, out_hbm, zbuf, dstv, ewv, acc, sem):
    cid = lax.axis_index("c")
    sid = lax.axis_index("s")
    wid = sid * NC + cid
    zeros = jnp.zeros((16,), jnp.float32)
    for i in range(40):
        zbuf[pl.ds(i * 16, 16)] = zeros
    pltpu.sync_copy(zbuf, acc.at[pl.ds(sid * 640, 640)])
    plsc.subcore_barrier()
    pltpu.sync_copy(ei_hbm.at[1, wid], dstv)
    pltpu.sync_copy(ew_hbm.at[wid], ewv)

    def fire(j, carry):
        pltpu.async_copy(ewv.at[j], acc.at[dstv.at[j]], sem, add=True)
        return carry

    lax.fori_loop(0, RPW, fire, 0)

    def drain(j, carry):
        pltpu.make_async_copy(ewv.at[j], acc.at[dstv.at[j]], sem).wait()
        return carry

    lax.fori_loop(0, RPW, drain, 0)
    plsc.subcore_barrier()

    @pl.when(sid == 0)
    def _():
        pltpu.sync_copy(acc, out_hbm.at[cid])


def _g_body(x_ref, wt_ref, deg_ref, g_ref):
    row = deg_ref[0:1, :] + deg_ref[1:2, :] + 1.0
    dinv = jnp.transpose(jnp.where(row > 0, lax.rsqrt(row), 0.0), (1, 0))
    h = jnp.dot(x_ref[...], wt_ref[...], preferred_element_type=jnp.float32)
    g_ref[...] = (dinv[:N] * h).astype(jnp.bfloat16)


_g_call = pl.pallas_call(
    _g_body,
    out_shape=jax.ShapeDtypeStruct((N, H1), jnp.bfloat16),
)


def _scale_rows(buf, ewv, j):
    def scale_one(e, w):
        ws = jnp.full((16,), w, jnp.float32)
        wb = plsc.pack(ws, ws, format=plsc.PackFormat.INTERLEAVED)
        for k in range(H1 // 32):
            sl = pl.ds(k * 32, 32)
            buf[e, sl] = buf[e, sl] * wb

    for off in (0, 16, 32, 48, 64, 80):
        wv = ewv[j, pl.ds(off, 16)]
        for e16 in range(16):
            scale_one(off + e16, wv[e16])
    wv = ewv[j, pl.ds(84, 16)]
    for e in range(96, ROW):
        scale_one(e, wv[e - 84])


@functools.partial(
    pl.kernel,
    out_type=jax.ShapeDtypeStruct((NC, NPAD, H1), jnp.bfloat16),
    mesh=_sc_mesh,
    compiler_params=_sc_params,
    scratch_types=[
        pltpu.VMEM((16, H1), jnp.bfloat16),
        pltpu.VMEM((RPW, ROW), jnp.int32),
        pltpu.VMEM((RPW, ROW), jnp.int32),
        pltpu.VMEM((RPW, ROW), jnp.float32),
        pltpu.VMEM((ROW, H1), jnp.bfloat16),
        pltpu.VMEM((ROW, H1), jnp.bfloat16),
        pltpu.VMEM((ROW, H1), jnp.bfloat16),
        pltpu.VMEM((ROW, H1), jnp.bfloat16),
        pltpu.VMEM_SHARED((NPAD, H1), jnp.bfloat16),
        pltpu.SemaphoreType.DMA,
        pltpu.SemaphoreType.DMA,
        pltpu.SemaphoreType.DMA,
        pltpu.SemaphoreType.DMA,
        pltpu.SemaphoreType.DMA,
        pltpu.SemaphoreType.DMA,
        pltpu.SemaphoreType.DMA,
        pltpu.SemaphoreType.DMA,
    ],
)
def _agg_kernel(ei_hbm, ew_hbm, g_hbm, out_hbm,
                zb, srcv, dstv, ewv, b0, b1, b2, b3, acc,
                sg0, sg1, sg2, sg3, ss0, ss1, ss2, ss3):
    cid = lax.axis_index("c")
    sid = lax.axis_index("s")
    wid = sid * NC + cid
    zeros = jnp.zeros((32,), jnp.bfloat16)
    for r in range(16):
        for k in range(H1 // 32):
            zb[r, pl.ds(k * 32, 32)] = zeros
    for k in range(40):
        pltpu.sync_copy(zb, acc.at[pl.ds(sid * 640 + k * 16, 16)])
    plsc.subcore_barrier()
    pltpu.sync_copy(ei_hbm.at[0, wid], srcv)
    pltpu.sync_copy(ei_hbm.at[1, wid], dstv)
    pltpu.sync_copy(ew_hbm.at[wid], ewv)

    bufs = [b0, b1, b2, b3]
    sgs = [sg0, sg1, sg2, sg3]
    sss = [ss0, ss1, ss2, ss3]
    NIT = RPW // 4

    for k in range(3):
        pltpu.async_copy(g_hbm.at[srcv.at[k]], bufs[k], sgs[k])

    def body(i, carry):
        for k in range(4):
            r = 4 * i + k
            pltpu.make_async_copy(g_hbm.at[srcv.at[r]], bufs[k], sgs[k]).wait()
            _scale_rows(bufs[k], ewv, r)
            pltpu.async_copy(bufs[k], acc.at[dstv.at[r]], sss[k], add=True)
            q = (k + 3) % 4
            nxt = r + 3
            if k == 0:
                @pl.when(i > 0)
                def _():
                    pltpu.make_async_copy(bufs[q], acc.at[dstv.at[0]],
                                          sss[q]).wait()
                pltpu.async_copy(g_hbm.at[srcv.at[nxt]], bufs[q], sgs[q])
            else:
                pltpu.make_async_copy(bufs[q], acc.at[dstv.at[0]],
                                      sss[q]).wait()

                @pl.when(i < NIT - 1)
                def _():
                    pltpu.async_copy(g_hbm.at[srcv.at[nxt]], bufs[q], sgs[q])
        return carry

    lax.fori_loop(0, NIT, body, 0)
    pltpu.make_async_copy(bufs[3], acc.at[dstv.at[0]], sss[3]).wait()
    plsc.subcore_barrier()
    pltpu.sync_copy(acc.at[pl.ds(sid * 640, 640)],
                    out_hbm.at[cid, pl.ds(sid * 640, 640)])


def _head_body(part_ref, g_ref, deg_ref, bg_ref, gam_ref, bet_ref,
               w1t_ref, b1_ref, w2t_ref, b2_ref, out_ref):
    row = deg_ref[0:1, :] + deg_ref[1:2, :] + 1.0
    dinv = jnp.transpose(jnp.where(row > 0, lax.rsqrt(row), 0.0), (1, 0))[:N]
    p = (part_ref[0, :N, :].astype(jnp.float32)
         + part_ref[1, :N, :].astype(jnp.float32))
    agg = dinv * (p + g_ref[...].astype(jnp.float32))
    t = jnp.maximum(agg + bg_ref[...], 0.0)
    mu = jnp.mean(t, axis=1, keepdims=True)
    d = t - mu
    var = jnp.mean(d * d, axis=1, keepdims=True)
    ln = d * lax.rsqrt(var + 1e-5) * gam_ref[...] + bet_ref[...]
    pooled = jnp.sum(ln, axis=0, keepdims=True) * (1.0 / N)
    z = jnp.maximum(
        jnp.dot(pooled, w1t_ref[...], preferred_element_type=jnp.float32)
        + b1_ref[...], 0.0)
    out_ref[...] = (jnp.dot(z, w2t_ref[...], preferred_element_type=jnp.float32)
                    + b2_ref[...])


def _make_head(a_dim):
    return pl.pallas_call(
        _head_body,
        out_shape=jax.ShapeDtypeStruct((1, a_dim), jnp.float32),
    )


def kernel(x, edge_index, edge_weight, W_gcn, b_gcn, gamma, beta, W1, b1, W2, b2):
    ei4 = edge_index.reshape(2, NW, RPW, ROW)
    ew3 = edge_weight.reshape(NW, RPW, ROW)

    deg_p = _deg_kernel(ei4, ew3)
    g = _g_call(x, W_gcn.T, deg_p)
    part = _agg_kernel(ei4, ew3, g)
    head = _make_head(W2.shape[0])
    return head(part, g, deg_p, b_gcn.reshape(1, H1), gamma.reshape(1, H1),
                beta.reshape(1, H1), W1.T, b1.reshape(1, -1), W2.T,
                b2.reshape(1, -1))
